# baseline (speedup 1.0000x reference)
import jax
import jax.numpy as jnp
from jax.experimental import pallas as pl
from jax.experimental.pallas import tpu as pltpu

_DIM = 768
_INTER = 256
_N_SHARED = 1
_N_ROUTING = 8
_TOPK = 2
_N_EXPERTS = _N_SHARED + _N_ROUTING
_T = 2048


def _moe_expert_kernel(x_ref, gw_ref, bias_ref, w1_ref, w2_ref, w3_ref, o_ref,
                       x16s, wvec_s):
    e = pl.program_id(0)

    @pl.when(e == 0)
    def _first_step():
        x16s[...] = x_ref[...].astype(jnp.bfloat16)

        scores = jax.lax.dot_general(
            x_ref[...], gw_ref[...], (((1,), (1,)), ((), ())),
            preferred_element_type=jnp.float32)
        scores = scores - jnp.max(scores, axis=-1, keepdims=True)
        es = jnp.exp(scores)
        p = es / jnp.sum(es, axis=-1, keepdims=True)
        sel = p + bias_ref[...]

        lane = jax.lax.broadcasted_iota(jnp.int32, sel.shape, 1)
        big = jnp.int32(_N_ROUTING + 1)

        m1 = jnp.max(sel, axis=-1, keepdims=True)
        i1 = jnp.min(jnp.where(sel >= m1, lane, big), axis=-1, keepdims=True)
        oh1 = (lane == i1).astype(jnp.float32)
        sel2 = sel - oh1 * jnp.float32(1e30)
        m2 = jnp.max(sel2, axis=-1, keepdims=True)
        i2 = jnp.min(jnp.where(sel2 >= m2, lane, big), axis=-1, keepdims=True)
        oh2 = (lane == i2).astype(jnp.float32)

        p1 = jnp.sum(p * oh1, axis=-1, keepdims=True)
        p2 = jnp.sum(p * oh2, axis=-1, keepdims=True)
        wvec_s[...] = p1 * oh1 + p2 * oh2

    x16 = x16s[...]
    w1e = w1_ref[0].astype(jnp.bfloat16)
    w3e = w3_ref[0].astype(jnp.bfloat16)
    w2e = w2_ref[0].astype(jnp.bfloat16)

    h1 = jnp.dot(x16, w1e, preferred_element_type=jnp.float32)
    h3 = jnp.dot(x16, w3e, preferred_element_type=jnp.float32)
    inter = jax.nn.silu(h1) * h3

    wv = wvec_s[...]
    ohe = (jax.lax.broadcasted_iota(jnp.int32, wv.shape, 1)
           == e - _N_SHARED).astype(jnp.float32)
    colv = jnp.sum(wv * ohe, axis=1, keepdims=True)
    scale = jnp.where(e < _N_SHARED, jnp.float32(1.0), colv)
    inter = inter * scale

    oe = jnp.dot(inter.astype(jnp.bfloat16), w2e,
                 preferred_element_type=jnp.float32)

    @pl.when(e == 0)
    def _init_out():
        o_ref[...] = oe

    @pl.when(e > 0)
    def _accum_out():
        o_ref[...] += oe


@jax.jit
def kernel(x, g_w, gate_bias, w1, w2, w3):
    Bb, Tt, C = x.shape
    x2 = x.reshape(Tt, C)
    bias2 = gate_bias.reshape(1, _N_ROUTING)

    out = pl.pallas_call(
        _moe_expert_kernel,
        grid=(_N_EXPERTS,),
        in_specs=[
            pl.BlockSpec((Tt, C), lambda e: (0, 0)),
            pl.BlockSpec((_N_ROUTING, C), lambda e: (0, 0)),
            pl.BlockSpec((1, _N_ROUTING), lambda e: (0, 0)),
            pl.BlockSpec((1, C, _INTER), lambda e: (e, 0, 0)),
            pl.BlockSpec((1, _INTER, C), lambda e: (e, 0, 0)),
            pl.BlockSpec((1, C, _INTER), lambda e: (e, 0, 0)),
        ],
        out_specs=pl.BlockSpec((Tt, C), lambda e: (0, 0)),
        out_shape=jax.ShapeDtypeStruct((Tt, C), jnp.float32),
        scratch_shapes=[
            pltpu.VMEM((Tt, C), jnp.bfloat16),
            pltpu.VMEM((Tt, _N_ROUTING), jnp.float32),
        ],
    )(x2, g_w, bias2, w1, w2, w3)
    return out.reshape(Bb, Tt, C)


# speedup vs baseline: 1.0148x; 1.0148x over previous
import jax
import jax.numpy as jnp
from jax.experimental import pallas as pl
from jax.experimental.pallas import tpu as pltpu

_DIM = 768
_INTER = 256
_N_SHARED = 1
_N_ROUTING = 8
_TOPK = 2
_N_EXPERTS = _N_SHARED + _N_ROUTING
_T = 2048
_EPG = 3
_N_STEPS = _N_EXPERTS // _EPG


def _moe_group_kernel(x_ref, gw_ref, bias_ref, w1_ref, w2_ref, w3_ref, o_ref,
                      x16s, wvec_s):
    s = pl.program_id(0)

    @pl.when(s == 0)
    def _first_step():
        x16s[...] = x_ref[...].astype(jnp.bfloat16)

        scores = jax.lax.dot_general(
            x_ref[...], gw_ref[...], (((1,), (1,)), ((), ())),
            preferred_element_type=jnp.float32)
        scores = scores - jnp.max(scores, axis=-1, keepdims=True)
        es = jnp.exp(scores)
        p = es / jnp.sum(es, axis=-1, keepdims=True)
        sel = p + bias_ref[...]

        lane = jax.lax.broadcasted_iota(jnp.int32, sel.shape, 1)
        big = jnp.int32(_N_ROUTING + 1)

        m1 = jnp.max(sel, axis=-1, keepdims=True)
        i1 = jnp.min(jnp.where(sel >= m1, lane, big), axis=-1, keepdims=True)
        oh1 = (lane == i1).astype(jnp.float32)
        sel2 = sel - oh1 * jnp.float32(1e30)
        m2 = jnp.max(sel2, axis=-1, keepdims=True)
        i2 = jnp.min(jnp.where(sel2 >= m2, lane, big), axis=-1, keepdims=True)
        oh2 = (lane == i2).astype(jnp.float32)

        p1 = jnp.sum(p * oh1, axis=-1, keepdims=True)
        p2 = jnp.sum(p * oh2, axis=-1, keepdims=True)
        wvec_s[...] = p1 * oh1 + p2 * oh2

    x16 = x16s[...]
    wv = wvec_s[...]
    lane8 = jax.lax.broadcasted_iota(jnp.int32, wv.shape, 1)

    osum = None
    for c in range(_EPG):
        w1e = w1_ref[c].astype(jnp.bfloat16)
        w3e = w3_ref[c].astype(jnp.bfloat16)
        w2e = w2_ref[c].astype(jnp.bfloat16)

        h1 = jnp.dot(x16, w1e, preferred_element_type=jnp.float32)
        h3 = jnp.dot(x16, w3e, preferred_element_type=jnp.float32)
        inter = jax.nn.silu(h1) * h3

        eid = s * _EPG + c
        ohe = (lane8 == eid - _N_SHARED).astype(jnp.float32)
        colv = jnp.sum(wv * ohe, axis=1, keepdims=True)
        scale = jnp.where(eid < _N_SHARED, jnp.float32(1.0), colv)
        inter = inter * scale

        oc = jnp.dot(inter.astype(jnp.bfloat16), w2e,
                     preferred_element_type=jnp.float32)
        osum = oc if osum is None else osum + oc

    @pl.when(s == 0)
    def _init_out():
        o_ref[...] = osum

    @pl.when(s > 0)
    def _accum_out():
        o_ref[...] += osum


@jax.jit
def kernel(x, g_w, gate_bias, w1, w2, w3):
    Bb, Tt, C = x.shape
    x2 = x.reshape(Tt, C)
    bias2 = gate_bias.reshape(1, _N_ROUTING)

    out = pl.pallas_call(
        _moe_group_kernel,
        grid=(_N_STEPS,),
        in_specs=[
            pl.BlockSpec((Tt, C), lambda s: (0, 0)),
            pl.BlockSpec((_N_ROUTING, C), lambda s: (0, 0)),
            pl.BlockSpec((1, _N_ROUTING), lambda s: (0, 0)),
            pl.BlockSpec((_EPG, C, _INTER), lambda s: (s, 0, 0)),
            pl.BlockSpec((_EPG, _INTER, C), lambda s: (s, 0, 0)),
            pl.BlockSpec((_EPG, C, _INTER), lambda s: (s, 0, 0)),
        ],
        out_specs=pl.BlockSpec((Tt, C), lambda s: (0, 0)),
        out_shape=jax.ShapeDtypeStruct((Tt, C), jnp.float32),
        scratch_shapes=[
            pltpu.VMEM((Tt, C), jnp.bfloat16),
            pltpu.VMEM((Tt, _N_ROUTING), jnp.float32),
        ],
    )(x2, g_w, bias2, w1, w2, w3)
    return out.reshape(Bb, Tt, C)


# speedup vs baseline: 1.1917x; 1.1744x over previous
import jax
import jax.numpy as jnp
from jax.experimental import pallas as pl
from jax.experimental.pallas import tpu as pltpu

_DIM = 768
_INTER = 256
_N_SHARED = 1
_N_ROUTING = 8
_TOPK = 2
_N_EXPERTS = _N_SHARED + _N_ROUTING
_BLK_T = 1024
_WIDE = _N_EXPERTS * _INTER


def _moe_block_kernel(x_ref, gw_ref, bias_ref, w1_ref, w2_ref, w3_ref, o_ref,
                      w13s, w2s):
    @pl.when(pl.program_id(0) == 0)
    def _cast_weights():
        for e in range(_N_EXPERTS):
            base = e * 2 * _INTER
            w13s[:, pl.ds(base, _INTER)] = w1_ref[e].astype(jnp.bfloat16)
            w13s[:, pl.ds(base + _INTER, _INTER)] = (
                w3_ref[e].astype(jnp.bfloat16))
            w2s[pl.ds(e * _INTER, _INTER), :] = w2_ref[e].astype(jnp.bfloat16)

    xb = x_ref[...]

    scores = jax.lax.dot_general(
        xb, gw_ref[...], (((1,), (1,)), ((), ())),
        preferred_element_type=jnp.float32)
    scores = scores - jnp.max(scores, axis=-1, keepdims=True)
    es = jnp.exp(scores)
    p = es / jnp.sum(es, axis=-1, keepdims=True)
    sel = p + bias_ref[...]

    lane = jax.lax.broadcasted_iota(jnp.int32, sel.shape, 1)
    big = jnp.int32(_N_ROUTING + 1)

    m1 = jnp.max(sel, axis=-1, keepdims=True)
    i1 = jnp.min(jnp.where(sel >= m1, lane, big), axis=-1, keepdims=True)
    oh1 = (lane == i1).astype(jnp.float32)
    sel2 = sel - oh1 * jnp.float32(1e30)
    m2 = jnp.max(sel2, axis=-1, keepdims=True)
    i2 = jnp.min(jnp.where(sel2 >= m2, lane, big), axis=-1, keepdims=True)
    oh2 = (lane == i2).astype(jnp.float32)

    p1 = jnp.sum(p * oh1, axis=-1, keepdims=True)
    p2 = jnp.sum(p * oh2, axis=-1, keepdims=True)
    wvec16 = (p1 * oh1 + p2 * oh2).astype(jnp.bfloat16)

    xb16 = xb.astype(jnp.bfloat16)
    pieces = []
    for e in range(_N_EXPERTS):
        he = jnp.dot(xb16, w13s[:, e * 2 * _INTER:(e + 1) * 2 * _INTER],
                     preferred_element_type=jnp.float32)
        h1e = he[:, :_INTER]
        h3e = he[:, _INTER:]
        ie = (jax.nn.silu(h1e) * h3e).astype(jnp.bfloat16)
        if e >= _N_SHARED:
            ie = ie * wvec16[:, e - _N_SHARED][:, None]
        pieces.append(ie)
    inter16 = jnp.concatenate(pieces, axis=1)

    o_ref[...] = jnp.dot(inter16, w2s[...], preferred_element_type=jnp.float32)


@jax.jit
def kernel(x, g_w, gate_bias, w1, w2, w3):
    Bb, Tt, C = x.shape
    x2 = x.reshape(Tt, C)
    bias2 = gate_bias.reshape(1, _N_ROUTING)

    grid = (Tt // _BLK_T,)
    out = pl.pallas_call(
        _moe_block_kernel,
        grid=grid,
        in_specs=[
            pl.BlockSpec((_BLK_T, C), lambda i: (i, 0)),
            pl.BlockSpec((_N_ROUTING, C), lambda i: (0, 0)),
            pl.BlockSpec((1, _N_ROUTING), lambda i: (0, 0)),
            pl.BlockSpec((_N_EXPERTS, C, _INTER), lambda i: (0, 0, 0)),
            pl.BlockSpec((_N_EXPERTS, _INTER, C), lambda i: (0, 0, 0)),
            pl.BlockSpec((_N_EXPERTS, C, _INTER), lambda i: (0, 0, 0)),
        ],
        out_specs=pl.BlockSpec((_BLK_T, C), lambda i: (i, 0)),
        out_shape=jax.ShapeDtypeStruct((Tt, C), jnp.float32),
        scratch_shapes=[
            pltpu.VMEM((_DIM, 2 * _WIDE), jnp.bfloat16),
            pltpu.VMEM((_WIDE, _DIM), jnp.bfloat16),
        ],
    )(x2, g_w, bias2, w1, w2, w3)
    return out.reshape(Bb, Tt, C)
